# baseline (speedup 1.0000x reference)
import functools

import jax
import jax.numpy as jnp
from jax import lax
from jax.experimental import pallas as pl
from jax.experimental.pallas import tpu as pltpu
from jax.experimental.pallas import tpu_sc as plsc

N = 10000
E = 320000
D = 128
NC = 2
NS = 16
NW = NC * NS
RPT = 640
NPAD = NS * RPT
DEGW = 128

DCHUNK = 128
DTOT = E // DCHUNK
DCPT = DTOT // NW
DEXTRA = DTOT - NW * DCPT

ACHUNK = 64
ATOT = E // ACHUNK
APT = ATOT // NW
AEXTRA = ATOT - NW * APT
NBUF = 4
NHALF = 3
HCPT = APT // NHALF
EMAIN = NW * APT * ACHUNK

_mesh = plsc.VectorSubcoreMesh(core_axis_name="c", subcore_axis_name="s")


@functools.partial(
    pl.kernel,
    out_type=jax.ShapeDtypeStruct((NC, NPAD, DEGW), jnp.float32),
    mesh=_mesh,
    scratch_types=[
        pltpu.VMEM((DCPT, DCHUNK), jnp.int32),
        pltpu.VMEM((DCHUNK,), jnp.int32),
        pltpu.VMEM((DCHUNK, DEGW), jnp.float32),
        pltpu.VMEM_SHARED((NPAD, DEGW), jnp.float32),
        pltpu.SemaphoreType.DMA,
    ],
)
def _deg_kernel(dst_hbm, dstx_hbm, ones_hbm, zeros_hbm, out_hbm,
                dst_v, dx_v, ones_v, acc, sem):
    cid = lax.axis_index("c")
    sid = lax.axis_index("s")
    wid = cid * NS + sid
    has_x = sid < DEXTRA // NC
    xrow = cid * (DEXTRA // NC) + sid

    pltpu.sync_copy(zeros_hbm, acc.at[pl.ds(sid * RPT, RPT)])
    pltpu.sync_copy(ones_hbm, ones_v)
    pltpu.sync_copy(dst_hbm.at[wid], dst_v)

    @pl.when(has_x)
    def _():
        pltpu.sync_copy(dstx_hbm.at[xrow], dx_v)

    plsc.subcore_barrier()

    def fire(j, carry):
        pltpu.async_copy(ones_v, acc.at[dst_v.at[j]], sem, add=True)
        return carry

    lax.fori_loop(0, DCPT, fire, 0)

    @pl.when(has_x)
    def _():
        pltpu.async_copy(ones_v, acc.at[dx_v], sem, add=True)

    def drain(j, carry):
        pltpu.make_async_copy(ones_v, acc.at[dst_v.at[j]], sem).wait()
        return carry

    lax.fori_loop(0, DCPT, drain, 0)

    @pl.when(has_x)
    def _():
        pltpu.make_async_copy(ones_v, acc.at[dx_v], sem).wait()

    plsc.subcore_barrier()
    pltpu.sync_copy(acc.at[pl.ds(sid * RPT, RPT)],
                    out_hbm.at[cid, pl.ds(sid * RPT, RPT)])


@functools.partial(
    pl.kernel,
    out_type=jax.ShapeDtypeStruct((NC, NPAD, D), jnp.float32),
    mesh=_mesh,
    scratch_types=(
        [pltpu.VMEM_SHARED((NPAD, D), jnp.float32)]
        + [pltpu.VMEM((HCPT, ACHUNK), jnp.int32)] * 2
        + [pltpu.VMEM((ACHUNK,), jnp.int32)] * 2
        + [pltpu.VMEM((ACHUNK, D), jnp.float32)] * NBUF
        + [pltpu.SemaphoreType.DMA] * (2 * NBUF)
    ),
)
def _agg_kernel(hs_hbm, src_hbm, dst_hbm, srcx_hbm, dstx_hbm, zrows_hbm,
                out_hbm, acc, src_v, dst_v, sx_v, dx_v, r0, r1, r2, r3,
                g0, g1, g2, g3, s0, s1, s2, s3):
    rows = (r0, r1, r2, r3)
    gsem = (g0, g1, g2, g3)
    ssem = (s0, s1, s2, s3)
    cid = lax.axis_index("c")
    sid = lax.axis_index("s")
    wid = cid * NS + sid
    has_x = sid < AEXTRA // NC
    xrow = cid * (AEXTRA // NC) + sid

    def fire_gather(j, b):
        pltpu.async_copy(hs_hbm.at[src_v.at[j]], rows[b], gsem[b])

    def wait_gather(j, b):
        pltpu.make_async_copy(hs_hbm.at[src_v.at[j]], rows[b], gsem[b]).wait()

    def fire_scatter(j, b):
        pltpu.async_copy(rows[b], acc.at[dst_v.at[j]], ssem[b], add=True)

    def wait_scatter(j, b):
        pltpu.make_async_copy(rows[b], acc.at[dst_v.at[j]], ssem[b]).wait()

    pltpu.sync_copy(zrows_hbm, acc.at[pl.ds(sid * RPT, RPT)])
    plsc.subcore_barrier()

    for h in range(NHALF):
        pltpu.sync_copy(src_hbm.at[wid, h], src_v)
        pltpu.sync_copy(dst_hbm.at[wid, h], dst_v)
        for c in range(NBUF - 1):
            fire_gather(c, c)

        def body(i, carry):
            for b in range(NBUF):
                j = i * NBUF + b
                jn = j + NBUF - 1
                bn = (b + NBUF - 1) % NBUF

                @pl.when(jnp.logical_and(j >= 1, jn < HCPT))
                def _():
                    wait_scatter(j - 1, bn)

                @pl.when(jn < HCPT)
                def _():
                    fire_gather(jn, bn)

                wait_gather(j, b)
                fire_scatter(j, b)
            return carry

        lax.fori_loop(0, HCPT // NBUF, body, 0)
        for b in range(NBUF):
            wait_scatter(HCPT - NBUF + b, (HCPT - NBUF + b) % NBUF)

    @pl.when(has_x)
    def _():
        pltpu.sync_copy(srcx_hbm.at[xrow], sx_v)
        pltpu.sync_copy(dstx_hbm.at[xrow], dx_v)
        pltpu.async_copy(hs_hbm.at[sx_v], rows[0], gsem[0]).wait()
        pltpu.sync_copy(rows[0], acc.at[dx_v], add=True)

    plsc.subcore_barrier()
    pltpu.sync_copy(acc.at[pl.ds(sid * RPT, RPT)],
                    out_hbm.at[cid, pl.ds(sid * RPT, RPT)])


def _tc0_body(x_ref, w1_ref, h_ref):
    h_ref[...] = jnp.dot(x_ref[...], w1_ref[...],
                         preferred_element_type=jnp.float32)


def _tc1_body(degp_ref, h_ref, dinv_ref, hs_ref):
    deg = degp_ref[0, :N, 0:1] + degp_ref[1, :N, 0:1] + 1.0
    dinv = lax.rsqrt(deg)
    dinv_ref[...] = dinv
    hs_ref[...] = h_ref[...] * dinv


def _tc2_body(p_ref, hs_ref, dinv_ref, b_ref, g_ref, be_ref, w2_ref, out_ref):
    dinv = dinv_ref[...]
    acc = p_ref[0, :N, :] + p_ref[1, :N, :] + hs_ref[...]
    y = acc * dinv + b_ref[...]
    mean = jnp.mean(y, axis=0, keepdims=True)
    var = jnp.mean((y - mean) * (y - mean), axis=0, keepdims=True)
    yn = (y - mean) * lax.rsqrt(var + 1e-5) * g_ref[...] + be_ref[...]
    yr = jnp.maximum(yn, 0.0)
    h2 = jnp.dot(yr, w2_ref[...], preferred_element_type=jnp.float32)
    out_ref[...] = h2 * dinv


def _tc3_body(p_ref, hs_ref, dinv_ref, b_ref, g_ref, be_ref, out_ref):
    acc = p_ref[0, :N, :] + p_ref[1, :N, :] + hs_ref[...]
    y = acc * dinv_ref[...] + b_ref[...]
    mean = jnp.mean(y, axis=0, keepdims=True)
    var = jnp.mean((y - mean) * (y - mean), axis=0, keepdims=True)
    out_ref[...] = (y - mean) * lax.rsqrt(var + 1e-5) * g_ref[...] + be_ref[...]


_f32 = jnp.float32
_tc0 = pl.pallas_call(
    _tc0_body,
    out_shape=jax.ShapeDtypeStruct((N, D), _f32),
)
_tc1 = pl.pallas_call(
    _tc1_body,
    out_shape=[jax.ShapeDtypeStruct((N, 1), _f32),
               jax.ShapeDtypeStruct((N, D), _f32)],
)
_tc2 = pl.pallas_call(
    _tc2_body,
    out_shape=jax.ShapeDtypeStruct((N, D), _f32),
)
_tc3 = pl.pallas_call(
    _tc3_body,
    out_shape=jax.ShapeDtypeStruct((N, D), _f32),
)


def kernel(e_prev, edge_index, W1, b1, gamma1, beta1, W2, b2, gamma2, beta2):
    src = edge_index[0]
    dst = edge_index[1]
    src_a = src[:EMAIN].reshape(NW, NHALF, HCPT, ACHUNK)
    dst_a = dst[:EMAIN].reshape(NW, NHALF, HCPT, ACHUNK)
    srcx = src[EMAIN:].reshape(AEXTRA, ACHUNK)
    dstx_a = dst[EMAIN:].reshape(AEXTRA, ACHUNK)
    dst_d = dst[:EMAIN].reshape(NW, DCPT, DCHUNK)
    dstx_d = dst[EMAIN:].reshape(DEXTRA, DCHUNK)

    ones_c = jnp.ones((DCHUNK, DEGW), _f32)
    zeros_r = jnp.zeros((RPT, DEGW), _f32)
    zrows = jnp.zeros((RPT, D), _f32)

    h1 = _tc0(e_prev, W1)
    degp = _deg_kernel(dst_d, dstx_d, ones_c, zeros_r)
    dinv, hs1 = _tc1(degp, h1)
    p1 = _agg_kernel(hs1, src_a, dst_a, srcx, dstx_a, zrows)
    hs2 = _tc2(p1, hs1, dinv, b1.reshape(1, D), gamma1.reshape(1, D),
               beta1.reshape(1, D), W2)
    p2 = _agg_kernel(hs2, src_a, dst_a, srcx, dstx_a, zrows)
    out = _tc3(p2, hs2, dinv, b2.reshape(1, D), gamma2.reshape(1, D),
               beta2.reshape(1, D))
    return out


# speedup vs baseline: 1.0618x; 1.0618x over previous
import functools

import jax
import jax.numpy as jnp
from jax import lax
from jax.experimental import pallas as pl
from jax.experimental.pallas import tpu as pltpu
from jax.experimental.pallas import tpu_sc as plsc

N = 10000
E = 320000
D = 128
NC = 2
NS = 16
NW = NC * NS
RPT = 640
NPAD = NS * RPT
DEGW = 128

CHUNK = 64
TOTC = E // CHUNK
CSMALL = 152
CBIG = 160
SN = 40
NSTG = 4
NBUF = 4

_mesh = plsc.VectorSubcoreMesh(core_axis_name="c", subcore_axis_name="s")


def _tile_geometry():
    cid = lax.axis_index("c")
    sid = lax.axis_index("s")
    w = sid * NC + cid
    big = jnp.logical_or(w < 16, w == NW - 1)
    wclip = jnp.minimum(w, 16)
    base = CBIG * wclip + CSMALL * (w - wclip)
    cnt = jnp.where(big, CBIG, CSMALL)
    return cid, sid, base, cnt


@functools.partial(
    pl.kernel,
    out_type=jax.ShapeDtypeStruct((NC, NPAD, DEGW), jnp.float32),
    mesh=_mesh,
    scratch_types=[
        pltpu.VMEM((SN, CHUNK), jnp.int32),
        pltpu.VMEM((CHUNK, DEGW), jnp.float32),
        pltpu.VMEM_SHARED((NPAD, DEGW), jnp.float32),
        pltpu.SemaphoreType.DMA,
    ],
)
def _deg_kernel(ei_hbm, ones_hbm, zeros_hbm, out_hbm, dst_v, ones_v, acc, sem):
    cid, sid, base, cnt = _tile_geometry()
    pltpu.sync_copy(zeros_hbm, acc.at[pl.ds(sid * RPT, RPT)])
    pltpu.sync_copy(ones_hbm, ones_v)
    plsc.subcore_barrier()

    for s in range(NSTG):
        row0 = pl.multiple_of(base + s * SN, 8)
        scnt = jnp.minimum(SN, cnt - s * SN)
        pltpu.sync_copy(ei_hbm.at[1, pl.ds(row0, SN)], dst_v)

        def fire(j, carry):
            pltpu.async_copy(ones_v, acc.at[dst_v.at[j]], sem, add=True)
            return carry

        lax.fori_loop(0, scnt, fire, 0)

        def drain(j, carry):
            pltpu.make_async_copy(ones_v, acc.at[dst_v.at[j]], sem).wait()
            return carry

        lax.fori_loop(0, scnt, drain, 0)

    plsc.subcore_barrier()
    pltpu.sync_copy(acc.at[pl.ds(sid * RPT, RPT)],
                    out_hbm.at[cid, pl.ds(sid * RPT, RPT)])


@functools.partial(
    pl.kernel,
    out_type=jax.ShapeDtypeStruct((NC, NPAD, D), jnp.float32),
    mesh=_mesh,
    scratch_types=(
        [pltpu.VMEM_SHARED((NPAD, D), jnp.float32)]
        + [pltpu.VMEM((SN, CHUNK), jnp.int32)] * 2
        + [pltpu.VMEM((CHUNK, D), jnp.float32)] * NBUF
        + [pltpu.SemaphoreType.DMA] * (2 * NBUF)
    ),
)
def _agg_kernel(hs_hbm, ei_hbm, zrows_hbm, out_hbm,
                acc, src_v, dst_v, r0, r1, r2, r3,
                g0, g1, g2, g3, s0, s1, s2, s3):
    rows = (r0, r1, r2, r3)
    gsem = (g0, g1, g2, g3)
    ssem = (s0, s1, s2, s3)
    cid, sid, base, cnt = _tile_geometry()

    def fire_gather(j, b):
        pltpu.async_copy(hs_hbm.at[src_v.at[j]], rows[b], gsem[b])

    def wait_gather(j, b):
        pltpu.make_async_copy(hs_hbm.at[src_v.at[j]], rows[b], gsem[b]).wait()

    def fire_scatter(j, b):
        pltpu.async_copy(rows[b], acc.at[dst_v.at[j]], ssem[b], add=True)

    def wait_scatter(j, b):
        pltpu.make_async_copy(rows[b], acc.at[dst_v.at[j]], ssem[b]).wait()

    pltpu.sync_copy(zrows_hbm, acc.at[pl.ds(sid * RPT, RPT)])
    plsc.subcore_barrier()

    for s in range(NSTG):
        row0 = pl.multiple_of(base + s * SN, 8)
        scnt = jnp.minimum(SN, cnt - s * SN)
        pltpu.sync_copy(ei_hbm.at[0, pl.ds(row0, SN)], src_v)
        pltpu.sync_copy(ei_hbm.at[1, pl.ds(row0, SN)], dst_v)
        for c in range(NBUF - 1):
            fire_gather(c, c)

        def body(i, carry):
            for b in range(NBUF):
                j = i * NBUF + b
                jn = j + NBUF - 1
                bn = (b + NBUF - 1) % NBUF

                @pl.when(jnp.logical_and(j >= 1, jn < scnt))
                def _():
                    wait_scatter(j - 1, bn)

                @pl.when(jn < scnt)
                def _():
                    fire_gather(jn, bn)

                @pl.when(j < scnt)
                def _():
                    wait_gather(j, b)
                    fire_scatter(j, b)
            return carry

        lax.fori_loop(0, SN // NBUF, body, 0)
        for b in range(NBUF):
            wait_scatter(scnt - NBUF + b, b)

    plsc.subcore_barrier()
    pltpu.sync_copy(acc.at[pl.ds(sid * RPT, RPT)],
                    out_hbm.at[cid, pl.ds(sid * RPT, RPT)])


def _tc0_body(x_ref, w1_ref, h_ref):
    h_ref[...] = jnp.dot(x_ref[...], w1_ref[...],
                         preferred_element_type=jnp.float32)


def _tc1_body(degp_ref, h_ref, dinv_ref, hs_ref):
    deg = degp_ref[0, :N, 0:1] + degp_ref[1, :N, 0:1] + 1.0
    dinv = lax.rsqrt(deg)
    dinv_ref[...] = dinv
    hs_ref[...] = h_ref[...] * dinv


def _tc2_body(p_ref, hs_ref, dinv_ref, b_ref, g_ref, be_ref, w2_ref, out_ref):
    dinv = dinv_ref[...]
    acc = p_ref[0, :N, :] + p_ref[1, :N, :] + hs_ref[...]
    y = acc * dinv + b_ref[...]
    mean = jnp.mean(y, axis=0, keepdims=True)
    var = jnp.mean((y - mean) * (y - mean), axis=0, keepdims=True)
    yn = (y - mean) * lax.rsqrt(var + 1e-5) * g_ref[...] + be_ref[...]
    yr = jnp.maximum(yn, 0.0)
    h2 = jnp.dot(yr, w2_ref[...], preferred_element_type=jnp.float32)
    out_ref[...] = h2 * dinv


def _tc3_body(p_ref, hs_ref, dinv_ref, b_ref, g_ref, be_ref, out_ref):
    acc = p_ref[0, :N, :] + p_ref[1, :N, :] + hs_ref[...]
    y = acc * dinv_ref[...] + b_ref[...]
    mean = jnp.mean(y, axis=0, keepdims=True)
    var = jnp.mean((y - mean) * (y - mean), axis=0, keepdims=True)
    out_ref[...] = (y - mean) * lax.rsqrt(var + 1e-5) * g_ref[...] + be_ref[...]


_f32 = jnp.float32
_tc0 = pl.pallas_call(
    _tc0_body,
    out_shape=jax.ShapeDtypeStruct((N, D), _f32),
)
_tc1 = pl.pallas_call(
    _tc1_body,
    out_shape=[jax.ShapeDtypeStruct((N, 1), _f32),
               jax.ShapeDtypeStruct((N, D), _f32)],
)
_tc2 = pl.pallas_call(
    _tc2_body,
    out_shape=jax.ShapeDtypeStruct((N, D), _f32),
)
_tc3 = pl.pallas_call(
    _tc3_body,
    out_shape=jax.ShapeDtypeStruct((N, D), _f32),
)


def kernel(e_prev, edge_index, W1, b1, gamma1, beta1, W2, b2, gamma2, beta2):
    ei3 = edge_index.reshape(2, TOTC, CHUNK)

    ones_c = jnp.ones((CHUNK, DEGW), _f32)
    zeros_r = jnp.zeros((RPT, DEGW), _f32)
    zrows = jnp.zeros((RPT, D), _f32)

    h1 = _tc0(e_prev, W1)
    degp = _deg_kernel(ei3, ones_c, zeros_r)
    dinv, hs1 = _tc1(degp, h1)
    p1 = _agg_kernel(hs1, ei3, zrows)
    hs2 = _tc2(p1, hs1, dinv, b1.reshape(1, D), gamma1.reshape(1, D),
               beta1.reshape(1, D), W2)
    p2 = _agg_kernel(hs2, ei3, zrows)
    out = _tc3(p2, hs2, dinv, b2.reshape(1, D), gamma2.reshape(1, D),
               beta2.reshape(1, D))
    return out
